# baseline (speedup 1.0000x reference)
import functools

import jax
import jax.numpy as jnp
from jax import lax
from jax.experimental import pallas as pl
from jax.experimental.pallas import tpu as pltpu
from jax.experimental.pallas import tpu_sc as plsc


def _transpose_table(tabT, V, D, NC, NS, mesh):
    NW = NC * NS
    RC = 2048
    n_full = V // RC
    tail = V - n_full * RC

    @functools.partial(
        pl.kernel,
        mesh=mesh,
        out_type=jax.ShapeDtypeStruct((V, D), jnp.float32),
        compiler_params=pltpu.CompilerParams(use_tc_tiling_on_sc=False, needs_layout_passes=False),
        scratch_types=[
            pltpu.VMEM((D, RC), jnp.float32),
            pltpu.VMEM((RC, D), jnp.float32),
        ],
    )
    def tr(tabT_hbm, tl_hbm, buf_v, tl_v):
        wid = lax.axis_index("s") * NC + lax.axis_index("c")
        iota = lax.iota(jnp.int32, 16)

        def do_chunk(r0, rc):
            pltpu.sync_copy(tabT_hbm.at[:, pl.ds(r0, rc)],
                            buf_v.at[:, pl.ds(0, rc)])

            def row_body(r, carry):
                lo = plsc.load_gather(buf_v, [iota, jnp.full((16,), 0, jnp.int32) + r])
                hi = plsc.load_gather(buf_v, [iota + 16, jnp.full((16,), 0, jnp.int32) + r])
                tl_v[r, pl.ds(0, 16)] = lo
                tl_v[r, pl.ds(16, 16)] = hi
                return carry

            lax.fori_loop(0, rc, row_body, 0)
            pltpu.sync_copy(tl_v.at[pl.ds(0, rc)], tl_hbm.at[pl.ds(r0, rc)])

        def chunk_body(k, carry):
            do_chunk((k * NW + wid) * RC, RC)
            return carry

        lax.fori_loop(0, n_full // NW, chunk_body, 0)
        k_extra = (n_full // NW) * NW + wid

        @pl.when(k_extra < n_full)
        def _():
            do_chunk(k_extra * RC, RC)

        if tail:
            @pl.when(wid == 0)
            def _():
                do_chunk(n_full * RC, tail)

    return tr(tabT)


def _lookup(tl, idx2, fw, wrep, brep, B, L, V, D, NC, NS, mesh):
    NW = NC * NS
    BT = 512
    n_blk = L * (B // BT)
    per_w = n_blk // NW
    NB = B * L * D

    @functools.partial(
        pl.kernel,
        mesh=mesh,
        out_type=jax.ShapeDtypeStruct((NB,), jnp.float32),
        compiler_params=pltpu.CompilerParams(use_tc_tiling_on_sc=False, needs_layout_passes=False),
        scratch_types=[
            pltpu.VMEM((BT,), jnp.int32),
            pltpu.VMEM((BT,), jnp.float32),
            pltpu.VMEM((BT, D), jnp.float32),
            pltpu.VMEM((D // 8, BT * 8), jnp.float32),
            pltpu.VMEM((D * 16,), jnp.float32),
            pltpu.VMEM((D * 16,), jnp.float32),
            pltpu.SemaphoreType.DMA,
            pltpu.SemaphoreType.DMA,
        ],
    )
    def lk(tl_hbm, idx_hbm, fw_hbm, wrep_hbm, brep_hbm, out_hbm,
           idx_v, fv_v, rows_v, chunk_v, w_v, b_v, sem_r, sem_f):
        wid = lax.axis_index("s") * NC + lax.axis_index("c")
        pltpu.sync_copy(wrep_hbm, w_v)
        pltpu.sync_copy(brep_hbm, b_v)
        iota = lax.iota(jnp.int32, 16)

        def blk_body(t, carry):
            j = t * NW + wid
            l = j // (B // BT)
            g = j % (B // BT)
            pltpu.sync_copy(idx_hbm.at[pl.ds(l * B + g * BT, BT)], idx_v)
            gr = pltpu.async_copy(tl_hbm.at[idx_v], rows_v, sem_r)
            gf = pltpu.async_copy(fw_hbm.at[idx_v], fv_v, sem_f)
            gr.wait()
            gf.wait()

            def u_body(u, carry2):
                r_idx = u * 16 + iota
                fvv = plsc.load_gather(fv_v, [r_idx])
                off = (u // 8) * (128 * 8) + (u % 8) * 16
                for dt in range(D // 8):
                    for ds in range(8):
                        d = dt * 8 + ds
                        src = plsc.load_gather(
                            rows_v, [r_idx, jnp.full((16,), d, jnp.int32)])
                        val = src + fvv * w_v[pl.ds(d * 16, 16)] + b_v[pl.ds(d * 16, 16)]
                        chunk_v[dt, pl.ds(off + ds * 128, 16)] = val
                return carry2

            lax.fori_loop(0, BT // 16, u_body, 0)
            for dt in range(D // 8):
                pltpu.sync_copy(
                    chunk_v.at[dt],
                    out_hbm.at[pl.ds((l * (D // 8) + dt) * (B * 8) + g * BT * 8,
                                     BT * 8)])
            return carry

        lax.fori_loop(0, per_w, blk_body, 0)

    return lk(tl, idx2, fw, wrep, brep)


def kernel(x, emb_table, freq_weights, freq_proj_w, freq_proj_b):
    B, L = x.shape
    V, D = emb_table.shape
    N = B * L

    info = plsc.get_sparse_core_info()
    NC, NS = info.num_cores, info.num_subcores
    mesh = plsc.VectorSubcoreMesh(core_axis_name="c", subcore_axis_name="s")

    tabT = jnp.transpose(emb_table)
    idx2 = jnp.transpose(x).reshape(N).astype(jnp.int32)
    wrep = jnp.repeat(0.1 * freq_proj_w[:, 0], 16).astype(jnp.float32)
    brep = jnp.repeat(0.1 * freq_proj_b, 16).astype(jnp.float32)

    tl = _transpose_table(tabT, V, D, NC, NS, mesh)
    o = _lookup(tl, idx2, freq_weights, wrep, brep, B, L, V, D, NC, NS, mesh)
    o5 = o.reshape(L, D // 8, B // 128, 8, 128)
    return jnp.transpose(o5, (2, 4, 0, 1, 3)).reshape(B, L, D)


# speedup vs baseline: 2.5378x; 2.5378x over previous
import functools

import jax
import jax.numpy as jnp
from jax import lax
from jax.experimental import pallas as pl
from jax.experimental.pallas import tpu as pltpu
from jax.experimental.pallas import tpu_sc as plsc


def _transpose_table(tabT, tail_flat, V, D, NC, NS, mesh):
    NW = NC * NS
    RC = 1024
    n_main = 976
    r512 = n_main * RC
    r64 = r512 + 512

    @functools.partial(
        pl.kernel,
        mesh=mesh,
        out_type=jax.ShapeDtypeStruct((V * D,), jnp.float32),
        compiler_params=pltpu.CompilerParams(
            use_tc_tiling_on_sc=True, needs_layout_passes=False),
        scratch_types=[
            pltpu.VMEM((D, RC), jnp.float32),
            pltpu.VMEM((RC * D,), jnp.float32),
            pltpu.VMEM((64 * D,), jnp.float32),
        ],
    )
    def tr(tabT_hbm, tail_hbm, tl_hbm, buf_v, tl_v, tt_v):
        wid = lax.axis_index("s") * NC + lax.axis_index("c")
        iota = lax.iota(jnp.int32, 16)

        def do_chunk(r0, rc):
            pltpu.sync_copy(tabT_hbm.at[:, pl.ds(r0, rc)],
                            buf_v.at[:, pl.ds(0, rc)])

            def row_body(r, carry):
                lo = plsc.load_gather(buf_v, [iota, jnp.full((16,), 0, jnp.int32) + r])
                hi = plsc.load_gather(buf_v, [iota + 16, jnp.full((16,), 0, jnp.int32) + r])
                tl_v[pl.ds(r * D, 16)] = lo
                tl_v[pl.ds(r * D + 16, 16)] = hi
                return carry

            lax.fori_loop(0, rc, row_body, 0, unroll=8)
            pltpu.sync_copy(tl_v.at[pl.ds(0, rc * D)],
                            tl_hbm.at[pl.ds(r0 * D, rc * D)])

        def chunk_body(t, carry):
            do_chunk((t * NW + wid) * RC, RC)
            return carry

        lax.fori_loop(0, n_main // NW, chunk_body, 0)
        k_extra = (n_main // NW) * NW + wid

        @pl.when(k_extra < n_main)
        def _():
            do_chunk(k_extra * RC, RC)

        @pl.when(wid == 16)
        def _():
            do_chunk(r512, 512)

        @pl.when(wid == 17)
        def _():
            pltpu.sync_copy(tail_hbm, tt_v)
            pltpu.sync_copy(tt_v, tl_hbm.at[pl.ds(r64 * D, 64 * D)])

    return tr(tabT, tail_flat)


def _lookup(tl, idx2, fw, wrep, brep, B, L, V, D, NC, NS, mesh):
    NW = NC * NS
    BT = 512
    n_blk = L * (B // BT)
    per_w = n_blk // NW
    NB = B * L * D

    @functools.partial(
        pl.kernel,
        mesh=mesh,
        out_type=jax.ShapeDtypeStruct((NB,), jnp.float32),
        compiler_params=pltpu.CompilerParams(
            use_tc_tiling_on_sc=False, needs_layout_passes=False),
        scratch_types=[
            pltpu.VMEM((BT,), jnp.int32),
            pltpu.VMEM((BT,), jnp.float32),
            pltpu.VMEM((BT, D), jnp.float32),
            pltpu.VMEM((D // 8, BT * 8), jnp.float32),
            pltpu.VMEM((D * 16,), jnp.float32),
            pltpu.VMEM((D * 16,), jnp.float32),
            pltpu.SemaphoreType.DMA,
            pltpu.SemaphoreType.DMA,
        ],
    )
    def lk(tl_hbm, idx_hbm, fw_hbm, wrep_hbm, brep_hbm, out_hbm,
           idx_v, fv_v, rows_v, chunk_v, w_v, b_v, sem_r, sem_f):
        wid = lax.axis_index("s") * NC + lax.axis_index("c")
        pltpu.sync_copy(wrep_hbm, w_v)
        pltpu.sync_copy(brep_hbm, b_v)
        iota = lax.iota(jnp.int32, 16)

        def blk_body(t, carry):
            j = t * NW + wid
            l = j // (B // BT)
            g = j % (B // BT)
            pltpu.sync_copy(idx_hbm.at[pl.ds(l * B + g * BT, BT)], idx_v)
            gr = pltpu.async_copy(tl_hbm.at[idx_v], rows_v, sem_r)
            gf = pltpu.async_copy(fw_hbm.at[idx_v], fv_v, sem_f)
            gr.wait()
            gf.wait()

            for dt in range(D // 8):
                wv = [w_v[pl.ds((dt * 8 + ds) * 16, 16)] for ds in range(8)]
                bv = [b_v[pl.ds((dt * 8 + ds) * 16, 16)] for ds in range(8)]

                def u_body(u, carry2):
                    r_idx = u * 16 + iota
                    fvv = plsc.load_gather(fv_v, [r_idx])
                    off = (u // 8) * (128 * 8) + (u % 8) * 16
                    for ds in range(8):
                        d = dt * 8 + ds
                        src = plsc.load_gather(
                            rows_v, [r_idx, jnp.full((16,), d, jnp.int32)])
                        chunk_v[dt, pl.ds(off + ds * 128, 16)] = (
                            src + fvv * wv[ds] + bv[ds])
                    return carry2

                lax.fori_loop(0, BT // 16, u_body, 0, unroll=2)
            for dt in range(D // 8):
                pltpu.sync_copy(
                    chunk_v.at[dt],
                    out_hbm.at[pl.ds((l * (D // 8) + dt) * (B * 8) + g * BT * 8,
                                     BT * 8)])
            return carry

        lax.fori_loop(0, per_w, blk_body, 0)

    return lk(tl, idx2, fw, wrep, brep)


def kernel(x, emb_table, freq_weights, freq_proj_w, freq_proj_b):
    B, L = x.shape
    V, D = emb_table.shape
    N = B * L

    info = plsc.get_sparse_core_info()
    NC, NS = info.num_cores, info.num_subcores
    mesh = plsc.VectorSubcoreMesh(core_axis_name="c", subcore_axis_name="s")

    tabT = jnp.transpose(emb_table)
    idx2 = jnp.transpose(x).reshape(N).astype(jnp.int32)
    wrep = jnp.repeat(0.1 * freq_proj_w[:, 0], 16).astype(jnp.float32)
    brep = jnp.repeat(0.1 * freq_proj_b, 16).astype(jnp.float32)

    tail_flat = emb_table[V - 64:].reshape(64 * D).astype(jnp.float32)
    tl_flat = _transpose_table(tabT, tail_flat, V, D, NC, NS, mesh)
    tl = tl_flat.reshape(V, D)
    o = _lookup(tl, idx2, freq_weights, wrep, brep, B, L, V, D, NC, NS, mesh)
    o5 = o.reshape(L, D // 8, B // 128, 8, 128)
    return jnp.transpose(o5, (2, 4, 0, 1, 3)).reshape(B, L, D)


# speedup vs baseline: 2.9021x; 1.1435x over previous
import functools

import jax
import jax.numpy as jnp
from jax import lax
from jax.experimental import pallas as pl
from jax.experimental.pallas import tpu as pltpu
from jax.experimental.pallas import tpu_sc as plsc


def _transpose_table(tabT, tail_flat, V, D, NC, NS, mesh):
    NW = NC * NS
    RC = 512
    n_main = 1952
    per_w = n_main // NW
    r512 = n_main * RC
    r64 = r512 + 512

    @functools.partial(
        pl.kernel,
        mesh=mesh,
        out_type=jax.ShapeDtypeStruct((V * D,), jnp.float32),
        compiler_params=pltpu.CompilerParams(
            use_tc_tiling_on_sc=True, needs_layout_passes=False),
        scratch_types=[
            pltpu.VMEM((D, RC), jnp.float32),
            pltpu.VMEM((D, RC), jnp.float32),
            pltpu.VMEM((RC * D,), jnp.float32),
            pltpu.VMEM((RC * D,), jnp.float32),
            pltpu.VMEM((64 * D,), jnp.float32),
            pltpu.SemaphoreType.DMA,
            pltpu.SemaphoreType.DMA,
            pltpu.SemaphoreType.DMA,
            pltpu.SemaphoreType.DMA,
        ],
    )
    def tr(tabT_hbm, tail_hbm, tl_hbm, b0, b1, t0, t1, tt_v,
           sem_i0, sem_i1, sem_w0, sem_w1):
        wid = lax.axis_index("s") * NC + lax.axis_index("c")
        iota = lax.iota(jnp.int32, 16)
        bufs = [b0, b1]
        tls = [t0, t1]
        isems = [sem_i0, sem_i1]
        wsems = [sem_w0, sem_w1]

        def chunk_r0(k):
            return (k * NW + wid) * RC

        def rd_issue(k, bi):
            pltpu.async_copy(tabT_hbm.at[:, pl.ds(chunk_r0(k), RC)],
                             bufs[bi], isems[bi])

        def rd_wait(bi):
            pltpu.make_async_copy(tabT_hbm.at[:, pl.ds(0, RC)],
                                  bufs[bi], isems[bi]).wait()

        def wr_issue(k, bi):
            pltpu.async_copy(tls[bi],
                             tl_hbm.at[pl.ds(chunk_r0(k) * D, RC * D)],
                             wsems[bi])

        def wr_drain(bi):
            pltpu.make_async_copy(tls[bi],
                                  tl_hbm.at[pl.ds(0, RC * D)],
                                  wsems[bi]).wait()

        def transpose(bi):
            def row_body(r, carry):
                lo = plsc.load_gather(
                    bufs[bi], [iota, jnp.full((16,), 0, jnp.int32) + r])
                hi = plsc.load_gather(
                    bufs[bi], [iota + 16, jnp.full((16,), 0, jnp.int32) + r])
                tls[bi][pl.ds(r * D, 16)] = lo
                tls[bi][pl.ds(r * D + 16, 16)] = hi
                return carry

            lax.fori_loop(0, RC, row_body, 0, unroll=8)

        rd_issue(0, 0)

        def body2(k2, carry):
            k0 = k2 * 2

            @pl.when(k0 + 1 < per_w)
            def _():
                rd_issue(k0 + 1, 1)

            rd_wait(0)

            @pl.when(k0 >= 2)
            def _():
                wr_drain(0)

            transpose(0)
            wr_issue(k0, 0)

            @pl.when(k0 + 1 < per_w)
            def _():
                @pl.when(k0 + 2 < per_w)
                def _():
                    rd_issue(k0 + 2, 0)

                rd_wait(1)

                @pl.when(k0 >= 1)
                def _():
                    wr_drain(1)

                transpose(1)
                wr_issue(k0 + 1, 1)

            return carry

        lax.fori_loop(0, (per_w + 1) // 2, body2, 0)
        wr_drain(0)
        wr_drain(1)

        @pl.when(wid == 0)
        def _():
            pltpu.sync_copy(tabT_hbm.at[:, pl.ds(r512, RC)], b0)

            def row_body(r, carry):
                lo = plsc.load_gather(
                    b0, [iota, jnp.full((16,), 0, jnp.int32) + r])
                hi = plsc.load_gather(
                    b0, [iota + 16, jnp.full((16,), 0, jnp.int32) + r])
                t0[pl.ds(r * D, 16)] = lo
                t0[pl.ds(r * D + 16, 16)] = hi
                return carry

            lax.fori_loop(0, RC, row_body, 0, unroll=8)
            pltpu.sync_copy(t0, tl_hbm.at[pl.ds(r512 * D, RC * D)])

        @pl.when(wid == 17)
        def _():
            pltpu.sync_copy(tail_hbm, tt_v)
            pltpu.sync_copy(tt_v, tl_hbm.at[pl.ds(r64 * D, 64 * D)])

    return tr(tabT, tail_flat)


def _lookup(tl, idx2, fw, wrep, brep, B, L, V, D, NC, NS, mesh):
    NW = NC * NS
    BT = 512
    GB = B // BT
    n_blk = L * GB
    per_w = n_blk // NW
    NB = B * L * D

    @functools.partial(
        pl.kernel,
        mesh=mesh,
        out_type=jax.ShapeDtypeStruct((NB,), jnp.float32),
        compiler_params=pltpu.CompilerParams(
            use_tc_tiling_on_sc=False, needs_layout_passes=False),
        scratch_types=[
            pltpu.VMEM((BT,), jnp.int32),
            pltpu.VMEM((BT,), jnp.int32),
            pltpu.VMEM((BT,), jnp.float32),
            pltpu.VMEM((BT,), jnp.float32),
            pltpu.VMEM((BT, D), jnp.float32),
            pltpu.VMEM((BT, D), jnp.float32),
            pltpu.VMEM((D // 8, BT * 8), jnp.float32),
            pltpu.VMEM((D // 8, BT * 8), jnp.float32),
            pltpu.VMEM((D * 16,), jnp.float32),
            pltpu.VMEM((D * 16,), jnp.float32),
            pltpu.SemaphoreType.DMA,
            pltpu.SemaphoreType.DMA,
            pltpu.SemaphoreType.DMA,
            pltpu.SemaphoreType.DMA,
            pltpu.SemaphoreType.DMA,
            pltpu.SemaphoreType.DMA,
        ],
    )
    def lk(tl_hbm, idx_hbm, fw_hbm, wrep_hbm, brep_hbm, out_hbm,
           i0, i1, f0, f1, r0, r1, c0, c1, w_v, b_v,
           sem_r0, sem_r1, sem_f0, sem_f1, sem_c0, sem_c1):
        wid = lax.axis_index("s") * NC + lax.axis_index("c")
        pltpu.sync_copy(wrep_hbm, w_v)
        pltpu.sync_copy(brep_hbm, b_v)
        iota = lax.iota(jnp.int32, 16)
        idxs = [i0, i1]
        fvs = [f0, f1]
        rows = [r0, r1]
        chunks = [c0, c1]
        rsems = [sem_r0, sem_r1]
        fsems = [sem_f0, sem_f1]
        csems = [sem_c0, sem_c1]

        def issue(t, bi):
            j = t * NW + wid
            l = j // GB
            g = j % GB
            pltpu.sync_copy(idx_hbm.at[pl.ds(l * B + g * BT, BT)], idxs[bi])
            pltpu.async_copy(tl_hbm.at[idxs[bi]], rows[bi], rsems[bi])
            pltpu.async_copy(fw_hbm.at[idxs[bi]], fvs[bi], fsems[bi])

        def wait_in(bi):
            pltpu.make_async_copy(tl_hbm.at[idxs[bi]], rows[bi], rsems[bi]).wait()
            pltpu.make_async_copy(fw_hbm.at[idxs[bi]], fvs[bi], fsems[bi]).wait()

        def drain_out(bi):
            for dt in range(D // 8):
                pltpu.make_async_copy(chunks[bi].at[dt],
                                      out_hbm.at[pl.ds(0, BT * 8)],
                                      csems[bi]).wait()

        def compute(t, bi):
            j = t * NW + wid
            l = j // GB
            g = j % GB
            for dt in range(D // 8):
                wv = [w_v[pl.ds((dt * 8 + ds) * 16, 16)] for ds in range(8)]
                bv = [b_v[pl.ds((dt * 8 + ds) * 16, 16)] for ds in range(8)]

                def u_body(u, carry2):
                    r_idx = u * 16 + iota
                    fvv = plsc.load_gather(fvs[bi], [r_idx])
                    off = (u // 8) * (128 * 8) + (u % 8) * 16
                    for ds in range(8):
                        d = dt * 8 + ds
                        src = plsc.load_gather(
                            rows[bi], [r_idx, jnp.full((16,), d, jnp.int32)])
                        chunks[bi][dt, pl.ds(off + ds * 128, 16)] = (
                            src + fvv * wv[ds] + bv[ds])
                    return carry2

                lax.fori_loop(0, BT // 16, u_body, 0, unroll=2)
            for dt in range(D // 8):
                pltpu.async_copy(
                    chunks[bi].at[dt],
                    out_hbm.at[pl.ds((l * (D // 8) + dt) * (B * 8) + g * BT * 8,
                                     BT * 8)],
                    csems[bi])

        issue(0, 0)

        def body2(t2, carry):
            t = t2 * 2
            issue(t + 1, 1)
            wait_in(0)

            @pl.when(t >= 2)
            def _():
                drain_out(0)

            compute(t, 0)

            @pl.when(t + 2 < per_w)
            def _():
                issue(t + 2, 0)

            wait_in(1)

            @pl.when(t >= 1)
            def _():
                drain_out(1)

            compute(t + 1, 1)
            return carry

        lax.fori_loop(0, per_w // 2, body2, 0)
        drain_out(0)
        drain_out(1)

    return lk(tl, idx2, fw, wrep, brep)


def kernel(x, emb_table, freq_weights, freq_proj_w, freq_proj_b):
    B, L = x.shape
    V, D = emb_table.shape
    N = B * L

    info = plsc.get_sparse_core_info()
    NC, NS = info.num_cores, info.num_subcores
    mesh = plsc.VectorSubcoreMesh(core_axis_name="c", subcore_axis_name="s")

    tabT = jnp.transpose(emb_table)
    idx2 = jnp.transpose(x).reshape(N).astype(jnp.int32)
    wrep = jnp.repeat(0.1 * freq_proj_w[:, 0], 16).astype(jnp.float32)
    brep = jnp.repeat(0.1 * freq_proj_b, 16).astype(jnp.float32)
    tail_flat = emb_table[V - 64:].reshape(64 * D).astype(jnp.float32)

    tl_flat = _transpose_table(tabT, tail_flat, V, D, NC, NS, mesh)
    tl = tl_flat.reshape(V, D)
    o = _lookup(tl, idx2, freq_weights, wrep, brep, B, L, V, D, NC, NS, mesh)
    o5 = o.reshape(L, D // 8, B // 128, 8, 128)
    return jnp.transpose(o5, (2, 4, 0, 1, 3)).reshape(B, L, D)


# speedup vs baseline: 4.0442x; 1.3936x over previous
import functools

import jax
import jax.numpy as jnp
from jax import lax
from jax.experimental import pallas as pl
from jax.experimental.pallas import tpu as pltpu
from jax.experimental.pallas import tpu_sc as plsc


def _transpose_table(tabT, tail_flat, V, D, NC, NS, mesh):
    NW = NC * NS
    RC = 512
    n_main = 1952
    per_w = n_main // NW
    r512 = n_main * RC
    r64 = r512 + 512

    @functools.partial(
        pl.kernel,
        mesh=mesh,
        out_type=jax.ShapeDtypeStruct((V * D,), jnp.float32),
        compiler_params=pltpu.CompilerParams(
            use_tc_tiling_on_sc=True, needs_layout_passes=False),
        scratch_types=[
            pltpu.VMEM((D, RC), jnp.float32),
            pltpu.VMEM((D, RC), jnp.float32),
            pltpu.VMEM((D * (RC + 1),), jnp.float32),
            pltpu.VMEM((RC * D,), jnp.float32),
            pltpu.VMEM((RC * D,), jnp.float32),
            pltpu.VMEM((64 * D,), jnp.float32),
            pltpu.SemaphoreType.DMA,
            pltpu.SemaphoreType.DMA,
            pltpu.SemaphoreType.DMA,
            pltpu.SemaphoreType.DMA,
        ],
    )
    def tr(tabT_hbm, tail_hbm, tl_hbm, b0, b1, bx, t0, t1, tt_v,
           sem_i0, sem_i1, sem_w0, sem_w1):
        wid = lax.axis_index("s") * NC + lax.axis_index("c")
        iota = lax.iota(jnp.int32, 16)
        stride = RC + 1
        ilo = iota * stride
        ihi = ilo + 16 * stride
        bufs = [b0, b1]
        tls = [t0, t1]
        isems = [sem_i0, sem_i1]
        wsems = [sem_w0, sem_w1]

        def chunk_r0(k):
            return (k * NW + wid) * RC

        def rd_issue(k, bi):
            pltpu.async_copy(tabT_hbm.at[:, pl.ds(chunk_r0(k), RC)],
                             bufs[bi], isems[bi])

        def rd_wait(bi):
            pltpu.make_async_copy(tabT_hbm.at[:, pl.ds(0, RC)],
                                  bufs[bi], isems[bi]).wait()

        def wr_issue(k, bi):
            pltpu.async_copy(tls[bi],
                             tl_hbm.at[pl.ds(chunk_r0(k) * D, RC * D)],
                             wsems[bi])

        def wr_drain(bi):
            pltpu.make_async_copy(tls[bi],
                                  tl_hbm.at[pl.ds(0, RC * D)],
                                  wsems[bi]).wait()

        def transpose(bi):
            def rp_body(c, carry):
                for rg in range(RC // 16):
                    bx[pl.ds(c * stride + rg * 16, 16)] = (
                        bufs[bi][c, pl.ds(rg * 16, 16)])
                return carry

            lax.fori_loop(0, D, rp_body, 0)

            def row_body(r, carry):
                tls[bi][pl.ds(r * D, 16)] = plsc.load_gather(bx, [ilo + r])
                tls[bi][pl.ds(r * D + 16, 16)] = plsc.load_gather(bx, [ihi + r])
                return carry

            lax.fori_loop(0, RC, row_body, 0, unroll=8)

        rd_issue(0, 0)

        def body2(k2, carry):
            k0 = k2 * 2

            @pl.when(k0 + 1 < per_w)
            def _():
                rd_issue(k0 + 1, 1)

            rd_wait(0)

            @pl.when(k0 >= 2)
            def _():
                wr_drain(0)

            transpose(0)
            wr_issue(k0, 0)

            @pl.when(k0 + 1 < per_w)
            def _():
                @pl.when(k0 + 2 < per_w)
                def _():
                    rd_issue(k0 + 2, 0)

                rd_wait(1)

                @pl.when(k0 >= 1)
                def _():
                    wr_drain(1)

                transpose(1)
                wr_issue(k0 + 1, 1)

            return carry

        lax.fori_loop(0, (per_w + 1) // 2, body2, 0)
        wr_drain(0)
        wr_drain(1)

        @pl.when(wid == 0)
        def _():
            pltpu.sync_copy(tabT_hbm.at[:, pl.ds(r512, RC)], b0)
            transpose(0)
            pltpu.sync_copy(t0, tl_hbm.at[pl.ds(r512 * D, RC * D)])

        @pl.when(wid == 17)
        def _():
            pltpu.sync_copy(tail_hbm, tt_v)
            pltpu.sync_copy(tt_v, tl_hbm.at[pl.ds(r64 * D, 64 * D)])

    return tr(tabT, tail_flat)


def _lookup(tl, idx2, fw, wrep, brep, B, L, V, D, NC, NS, mesh):
    NW = NC * NS
    BT = 512
    GB = B // BT
    n_blk = L * GB
    per_w = n_blk // NW
    NB = B * L * D

    @functools.partial(
        pl.kernel,
        mesh=mesh,
        out_type=jax.ShapeDtypeStruct((NB,), jnp.float32),
        compiler_params=pltpu.CompilerParams(
            use_tc_tiling_on_sc=False, needs_layout_passes=False),
        scratch_types=[
            pltpu.VMEM((BT,), jnp.int32),
            pltpu.VMEM((BT,), jnp.int32),
            pltpu.VMEM((BT,), jnp.float32),
            pltpu.VMEM((BT,), jnp.float32),
            pltpu.VMEM((BT, D), jnp.float32),
            pltpu.VMEM((BT, D), jnp.float32),
            pltpu.VMEM((BT * (D + 1),), jnp.float32),
            pltpu.VMEM((D // 8, BT * 8), jnp.float32),
            pltpu.VMEM((D // 8, BT * 8), jnp.float32),
            pltpu.VMEM((D * 16,), jnp.float32),
            pltpu.VMEM((D * 16,), jnp.float32),
            pltpu.SemaphoreType.DMA,
            pltpu.SemaphoreType.DMA,
            pltpu.SemaphoreType.DMA,
            pltpu.SemaphoreType.DMA,
            pltpu.SemaphoreType.DMA,
            pltpu.SemaphoreType.DMA,
        ],
    )
    def lk(tl_hbm, idx_hbm, fw_hbm, wrep_hbm, brep_hbm, out_hbm,
           i0, i1, f0, f1, r0, r1, rx, c0, c1, w_v, b_v,
           sem_r0, sem_r1, sem_f0, sem_f1, sem_c0, sem_c1):
        wid = lax.axis_index("s") * NC + lax.axis_index("c")
        pltpu.sync_copy(wrep_hbm, w_v)
        pltpu.sync_copy(brep_hbm, b_v)
        iota = lax.iota(jnp.int32, 16)
        idxs = [i0, i1]
        fvs = [f0, f1]
        rows = [r0, r1]
        chunks = [c0, c1]
        rsems = [sem_r0, sem_r1]
        fsems = [sem_f0, sem_f1]
        csems = [sem_c0, sem_c1]

        def issue(t, bi):
            j = t * NW + wid
            l = j // GB
            g = j % GB
            pltpu.sync_copy(idx_hbm.at[pl.ds(l * B + g * BT, BT)], idxs[bi])
            pltpu.async_copy(tl_hbm.at[idxs[bi]], rows[bi], rsems[bi])
            pltpu.async_copy(fw_hbm.at[idxs[bi]], fvs[bi], fsems[bi])

        def wait_in(bi):
            pltpu.make_async_copy(tl_hbm.at[idxs[bi]], rows[bi],
                                  rsems[bi]).wait()
            pltpu.make_async_copy(fw_hbm.at[idxs[bi]], fvs[bi], fsems[bi]).wait()

        def drain_out(bi):
            for dt in range(D // 8):
                pltpu.make_async_copy(chunks[bi].at[dt],
                                      out_hbm.at[pl.ds(0, BT * 8)],
                                      csems[bi]).wait()

        stride = D + 1
        i33 = iota * stride

        def compute(t, bi):
            j = t * NW + wid
            l = j // GB
            g = j % GB

            def rp_body(r, carry):
                rx[pl.ds(r * stride, 16)] = rows[bi][r, pl.ds(0, 16)]
                rx[pl.ds(r * stride + 16, 16)] = rows[bi][r, pl.ds(16, 16)]
                return carry

            lax.fori_loop(0, BT, rp_body, 0, unroll=8)
            for dt in range(D // 8):
                wv = [w_v[pl.ds((dt * 8 + ds) * 16, 16)] for ds in range(8)]
                bv = [b_v[pl.ds((dt * 8 + ds) * 16, 16)] for ds in range(8)]

                def u_body(u, carry2):
                    r_idx = u * 16 + iota
                    fvv = plsc.load_gather(fvs[bi], [r_idx])
                    rbase = u * (16 * stride)
                    off = (u // 8) * (128 * 8) + (u % 8) * 16
                    for ds in range(8):
                        d = dt * 8 + ds
                        src = plsc.load_gather(rx, [i33 + (rbase + d)])
                        chunks[bi][dt, pl.ds(off + ds * 128, 16)] = (
                            src + fvv * wv[ds] + bv[ds])
                    return carry2

                lax.fori_loop(0, BT // 16, u_body, 0, unroll=2)
            for dt in range(D // 8):
                pltpu.async_copy(
                    chunks[bi].at[dt],
                    out_hbm.at[pl.ds((l * (D // 8) + dt) * (B * 8) + g * BT * 8,
                                     BT * 8)],
                    csems[bi])

        issue(0, 0)

        def body2(t2, carry):
            t = t2 * 2
            issue(t + 1, 1)
            wait_in(0)

            @pl.when(t >= 2)
            def _():
                drain_out(0)

            compute(t, 0)

            @pl.when(t + 2 < per_w)
            def _():
                issue(t + 2, 0)

            wait_in(1)

            @pl.when(t >= 1)
            def _():
                drain_out(1)

            compute(t + 1, 1)
            return carry

        lax.fori_loop(0, per_w // 2, body2, 0)
        drain_out(0)
        drain_out(1)

    return lk(tl, idx2, fw, wrep, brep)


def kernel(x, emb_table, freq_weights, freq_proj_w, freq_proj_b):
    B, L = x.shape
    V, D = emb_table.shape
    N = B * L

    info = plsc.get_sparse_core_info()
    NC, NS = info.num_cores, info.num_subcores
    mesh = plsc.VectorSubcoreMesh(core_axis_name="c", subcore_axis_name="s")

    tabT = jnp.transpose(emb_table)
    idx2 = jnp.transpose(x).reshape(N).astype(jnp.int32)
    wrep = jnp.repeat(0.1 * freq_proj_w[:, 0], 16).astype(jnp.float32)
    brep = jnp.repeat(0.1 * freq_proj_b, 16).astype(jnp.float32)
    tail_flat = emb_table[V - 64:].reshape(64 * D).astype(jnp.float32)

    tl_flat = _transpose_table(tabT, tail_flat, V, D, NC, NS, mesh)
    tl = tl_flat.reshape(V, D)
    o = _lookup(tl, idx2, freq_weights, wrep, brep, B, L, V, D, NC, NS, mesh)
    o5 = o.reshape(L, D // 8, B // 128, 8, 128)
    return jnp.transpose(o5, (2, 4, 0, 1, 3)).reshape(B, L, D)


# speedup vs baseline: 4.1601x; 1.0287x over previous
import functools

import jax
import jax.numpy as jnp
from jax import lax
from jax.experimental import pallas as pl
from jax.experimental.pallas import tpu as pltpu
from jax.experimental.pallas import tpu_sc as plsc


def _transpose_table(tabT, tail_flat, V, D, NC, NS, mesh):
    NW = NC * NS
    RC = 512
    n_main = 1952
    per_w = n_main // NW
    r512 = n_main * RC
    r64 = r512 + 512

    @functools.partial(
        pl.kernel,
        mesh=mesh,
        out_type=jax.ShapeDtypeStruct((V * D,), jnp.float32),
        compiler_params=pltpu.CompilerParams(
            use_tc_tiling_on_sc=True, needs_layout_passes=False),
        scratch_types=[
            pltpu.VMEM((D, RC), jnp.float32),
            pltpu.VMEM((D, RC), jnp.float32),
            pltpu.VMEM((D * (RC + 1),), jnp.float32),
            pltpu.VMEM((RC * D,), jnp.float32),
            pltpu.VMEM((RC * D,), jnp.float32),
            pltpu.VMEM((64 * D,), jnp.float32),
            pltpu.SemaphoreType.DMA,
            pltpu.SemaphoreType.DMA,
            pltpu.SemaphoreType.DMA,
            pltpu.SemaphoreType.DMA,
        ],
    )
    def tr(tabT_hbm, tail_hbm, tl_hbm, b0, b1, bx, t0, t1, tt_v,
           sem_i0, sem_i1, sem_w0, sem_w1):
        wid = lax.axis_index("s") * NC + lax.axis_index("c")
        iota = lax.iota(jnp.int32, 16)
        stride = RC + 1
        ilo = iota * stride
        ihi = ilo + 16 * stride
        bufs = [b0, b1]
        tls = [t0, t1]
        isems = [sem_i0, sem_i1]
        wsems = [sem_w0, sem_w1]

        def chunk_r0(k):
            return (k * NW + wid) * RC

        def rd_issue(k, bi):
            for ct in range(D // 8):
                pltpu.async_copy(
                    tabT_hbm.at[pl.ds(ct * 8, 8), pl.ds(chunk_r0(k), RC)],
                    bufs[bi].at[pl.ds(ct * 8, 8), :], isems[bi])

        def rd_wait(bi):
            for ct in range(D // 8):
                pltpu.make_async_copy(
                    tabT_hbm.at[pl.ds(0, 8), pl.ds(0, RC)],
                    bufs[bi].at[pl.ds(ct * 8, 8), :], isems[bi]).wait()

        def wr_issue(k, bi):
            pltpu.async_copy(tls[bi],
                             tl_hbm.at[pl.ds(chunk_r0(k) * D, RC * D)],
                             wsems[bi])

        def wr_drain(bi):
            pltpu.make_async_copy(tls[bi],
                                  tl_hbm.at[pl.ds(0, RC * D)],
                                  wsems[bi]).wait()

        def transpose(bi):
            def rp_body(c, carry):
                for rg in range(RC // 16):
                    bx[pl.ds(c * stride + rg * 16, 16)] = (
                        bufs[bi][c, pl.ds(rg * 16, 16)])
                return carry

            lax.fori_loop(0, D, rp_body, 0)

            def row_body(r, carry):
                tls[bi][pl.ds(r * D, 16)] = plsc.load_gather(bx, [ilo + r])
                tls[bi][pl.ds(r * D + 16, 16)] = plsc.load_gather(bx, [ihi + r])
                return carry

            lax.fori_loop(0, RC, row_body, 0, unroll=8)

        rd_issue(0, 0)

        def body2(k2, carry):
            k0 = k2 * 2

            @pl.when(k0 + 1 < per_w)
            def _():
                rd_issue(k0 + 1, 1)

            rd_wait(0)

            @pl.when(k0 >= 2)
            def _():
                wr_drain(0)

            transpose(0)
            wr_issue(k0, 0)

            @pl.when(k0 + 1 < per_w)
            def _():
                @pl.when(k0 + 2 < per_w)
                def _():
                    rd_issue(k0 + 2, 0)

                rd_wait(1)

                @pl.when(k0 >= 1)
                def _():
                    wr_drain(1)

                transpose(1)
                wr_issue(k0 + 1, 1)

            return carry

        lax.fori_loop(0, (per_w + 1) // 2, body2, 0)
        wr_drain(0)
        wr_drain(1)

        @pl.when(wid == 0)
        def _():
            pltpu.sync_copy(tabT_hbm.at[:, pl.ds(r512, RC)], b0)
            transpose(0)
            pltpu.sync_copy(t0, tl_hbm.at[pl.ds(r512 * D, RC * D)])

        @pl.when(wid == 17)
        def _():
            pltpu.sync_copy(tail_hbm, tt_v)
            pltpu.sync_copy(tt_v, tl_hbm.at[pl.ds(r64 * D, 64 * D)])

    return tr(tabT, tail_flat)


def _lookup(tl, idx2, fw, wrep, brep, B, L, V, D, NC, NS, mesh):
    NW = NC * NS
    BT = 512
    GB = B // BT
    n_blk = L * GB
    per_w = n_blk // NW
    NB = B * L * D

    @functools.partial(
        pl.kernel,
        mesh=mesh,
        out_type=jax.ShapeDtypeStruct((NB,), jnp.float32),
        compiler_params=pltpu.CompilerParams(
            use_tc_tiling_on_sc=False, needs_layout_passes=False),
        scratch_types=[
            pltpu.VMEM((BT,), jnp.int32),
            pltpu.VMEM((BT,), jnp.int32),
            pltpu.VMEM((BT,), jnp.float32),
            pltpu.VMEM((BT,), jnp.float32),
            pltpu.VMEM((BT, D), jnp.float32),
            pltpu.VMEM((BT, D), jnp.float32),
            pltpu.VMEM((BT * (D + 1),), jnp.float32),
            pltpu.VMEM((D // 8, BT * 8), jnp.float32),
            pltpu.VMEM((D // 8, BT * 8), jnp.float32),
            pltpu.VMEM((D * 16,), jnp.float32),
            pltpu.VMEM((D * 16,), jnp.float32),
            pltpu.SemaphoreType.DMA,
            pltpu.SemaphoreType.DMA,
            pltpu.SemaphoreType.DMA,
            pltpu.SemaphoreType.DMA,
            pltpu.SemaphoreType.DMA,
            pltpu.SemaphoreType.DMA,
            pltpu.SemaphoreType.DMA,
            pltpu.SemaphoreType.DMA,
        ],
    )
    def lk(tl_hbm, idx_hbm, fw_hbm, wrep_hbm, brep_hbm, out_hbm,
           i0, i1, f0, f1, r0, r1, rx, c0, c1, w_v, b_v,
           sem_x0, sem_x1, sem_r0, sem_r1, sem_f0, sem_f1, sem_c0, sem_c1):
        wid = lax.axis_index("s") * NC + lax.axis_index("c")
        pltpu.sync_copy(wrep_hbm, w_v)
        pltpu.sync_copy(brep_hbm, b_v)
        iota = lax.iota(jnp.int32, 16)
        idxs = [i0, i1]
        fvs = [f0, f1]
        rows = [r0, r1]
        chunks = [c0, c1]
        xsems = [sem_x0, sem_x1]
        rsems = [sem_r0, sem_r1]
        fsems = [sem_f0, sem_f1]
        csems = [sem_c0, sem_c1]

        def issue_idx(t, bi):
            j = t * NW + wid
            l = j // GB
            g = j % GB
            pltpu.async_copy(idx_hbm.at[pl.ds(l * B + g * BT, BT)],
                             idxs[bi], xsems[bi])

        def issue_gth(t, bi):
            pltpu.make_async_copy(idx_hbm.at[pl.ds(0, BT)], idxs[bi],
                                  xsems[bi]).wait()
            pltpu.async_copy(tl_hbm.at[idxs[bi]], rows[bi], rsems[bi])
            pltpu.async_copy(fw_hbm.at[idxs[bi]], fvs[bi], fsems[bi])

        def wait_in(bi):
            pltpu.make_async_copy(tl_hbm.at[idxs[bi]], rows[bi],
                                  rsems[bi]).wait()
            pltpu.make_async_copy(fw_hbm.at[idxs[bi]], fvs[bi], fsems[bi]).wait()

        def drain_out(bi):
            for dt in range(D // 8):
                pltpu.make_async_copy(chunks[bi].at[dt],
                                      out_hbm.at[pl.ds(0, BT * 8)],
                                      csems[bi]).wait()

        stride = D + 1
        i33 = iota * stride

        def compute(t, bi):
            j = t * NW + wid
            l = j // GB
            g = j % GB

            def rp_body(r, carry):
                rx[pl.ds(r * stride, 16)] = rows[bi][r, pl.ds(0, 16)]
                rx[pl.ds(r * stride + 16, 16)] = rows[bi][r, pl.ds(16, 16)]
                return carry

            lax.fori_loop(0, BT, rp_body, 0, unroll=8)
            for dt in range(D // 8):
                wv = [w_v[pl.ds((dt * 8 + ds) * 16, 16)] for ds in range(8)]
                bv = [b_v[pl.ds((dt * 8 + ds) * 16, 16)] for ds in range(8)]

                def u_body(u, carry2):
                    r_idx = u * 16 + iota
                    fvv = plsc.load_gather(fvs[bi], [r_idx])
                    rbase = u * (16 * stride)
                    off = (u // 8) * (128 * 8) + (u % 8) * 16
                    for ds in range(8):
                        d = dt * 8 + ds
                        src = plsc.load_gather(rx, [i33 + (rbase + d)])
                        chunks[bi][dt, pl.ds(off + ds * 128, 16)] = (
                            src + fvv * wv[ds] + bv[ds])
                    return carry2

                lax.fori_loop(0, BT // 16, u_body, 0, unroll=2)
            for dt in range(D // 8):
                pltpu.async_copy(
                    chunks[bi].at[dt],
                    out_hbm.at[pl.ds((l * (D // 8) + dt) * (B * 8) + g * BT * 8,
                                     BT * 8)],
                    csems[bi])

        issue_idx(0, 0)
        issue_idx(1, 1)
        issue_gth(0, 0)

        def body2(t2, carry):
            t = t2 * 2
            issue_gth(t + 1, 1)
            wait_in(0)

            @pl.when(t + 2 < per_w)
            def _():
                issue_idx(t + 2, 0)

            @pl.when(t >= 2)
            def _():
                drain_out(0)

            compute(t, 0)

            @pl.when(t + 2 < per_w)
            def _():
                issue_gth(t + 2, 0)

            wait_in(1)

            @pl.when(t + 3 < per_w)
            def _():
                issue_idx(t + 3, 1)

            @pl.when(t >= 1)
            def _():
                drain_out(1)

            compute(t + 1, 1)
            return carry

        lax.fori_loop(0, per_w // 2, body2, 0)
        drain_out(0)
        drain_out(1)

    return lk(tl, idx2, fw, wrep, brep)


def kernel(x, emb_table, freq_weights, freq_proj_w, freq_proj_b):
    B, L = x.shape
    V, D = emb_table.shape
    N = B * L

    info = plsc.get_sparse_core_info()
    NC, NS = info.num_cores, info.num_subcores
    mesh = plsc.VectorSubcoreMesh(core_axis_name="c", subcore_axis_name="s")

    tabT = jnp.transpose(emb_table)
    idx2 = jnp.transpose(x).reshape(N).astype(jnp.int32)
    wrep = jnp.repeat(0.1 * freq_proj_w[:, 0], 16).astype(jnp.float32)
    brep = jnp.repeat(0.1 * freq_proj_b, 16).astype(jnp.float32)
    tail_flat = emb_table[V - 64:].reshape(64 * D).astype(jnp.float32)

    tl_flat = _transpose_table(tabT, tail_flat, V, D, NC, NS, mesh)
    tl = tl_flat.reshape(V, D)
    o = _lookup(tl, idx2, freq_weights, wrep, brep, B, L, V, D, NC, NS, mesh)
    o5 = o.reshape(L, D // 8, B // 128, 8, 128)
    return jnp.transpose(o5, (2, 4, 0, 1, 3)).reshape(B, L, D)


# speedup vs baseline: 4.1663x; 1.0015x over previous
import functools

import jax
import jax.numpy as jnp
from jax import lax
from jax.experimental import pallas as pl
from jax.experimental.pallas import tpu as pltpu
from jax.experimental.pallas import tpu_sc as plsc


def _transpose_table(tabT, tail_flat, V, D, NC, NS, mesh):
    NW = NC * NS
    RC = 512
    n_main = 1952
    per_w = n_main // NW
    r512 = n_main * RC
    r64 = r512 + 512

    @functools.partial(
        pl.kernel,
        mesh=mesh,
        out_type=jax.ShapeDtypeStruct((V * D,), jnp.float32),
        compiler_params=pltpu.CompilerParams(
            use_tc_tiling_on_sc=True, needs_layout_passes=False),
        scratch_types=[
            pltpu.VMEM((D, RC), jnp.float32),
            pltpu.VMEM((D, RC), jnp.float32),
            pltpu.VMEM((D * (RC + 1),), jnp.float32),
            pltpu.VMEM((RC * D,), jnp.float32),
            pltpu.VMEM((RC * D,), jnp.float32),
            pltpu.VMEM((64 * D,), jnp.float32),
            pltpu.SemaphoreType.DMA,
            pltpu.SemaphoreType.DMA,
            pltpu.SemaphoreType.DMA,
            pltpu.SemaphoreType.DMA,
        ],
    )
    def tr(tabT_hbm, tail_hbm, tl_hbm, b0, b1, bx, t0, t1, tt_v,
           sem_i0, sem_i1, sem_w0, sem_w1):
        wid = lax.axis_index("s") * NC + lax.axis_index("c")
        iota = lax.iota(jnp.int32, 16)
        stride = RC + 1
        ilo = iota * stride
        ihi = ilo + 16 * stride
        bufs = [b0, b1]
        tls = [t0, t1]
        isems = [sem_i0, sem_i1]
        wsems = [sem_w0, sem_w1]

        def chunk_r0(k):
            return (k * NW + wid) * RC

        def rd_issue(k, bi):
            for ct in range(D // 8):
                pltpu.async_copy(
                    tabT_hbm.at[pl.ds(ct * 8, 8), pl.ds(chunk_r0(k), RC)],
                    bufs[bi].at[pl.ds(ct * 8, 8), :], isems[bi])

        def rd_wait(bi):
            for ct in range(D // 8):
                pltpu.make_async_copy(
                    tabT_hbm.at[pl.ds(0, 8), pl.ds(0, RC)],
                    bufs[bi].at[pl.ds(ct * 8, 8), :], isems[bi]).wait()

        def wr_issue(k, bi):
            pltpu.async_copy(tls[bi],
                             tl_hbm.at[pl.ds(chunk_r0(k) * D, RC * D)],
                             wsems[bi])

        def wr_drain(bi):
            pltpu.make_async_copy(tls[bi],
                                  tl_hbm.at[pl.ds(0, RC * D)],
                                  wsems[bi]).wait()

        def transpose(bi):
            def rp_body(c, carry):
                for rg in range(RC // 16):
                    bx[pl.ds(c * stride + rg * 16, 16)] = (
                        bufs[bi][c, pl.ds(rg * 16, 16)])
                return carry

            lax.fori_loop(0, D, rp_body, 0, unroll=4)

            def row_body(r, carry):
                tls[bi][pl.ds(r * D, 16)] = plsc.load_gather(bx, [ilo + r])
                tls[bi][pl.ds(r * D + 16, 16)] = plsc.load_gather(bx, [ihi + r])
                return carry

            lax.fori_loop(0, RC, row_body, 0, unroll=16)

        rd_issue(0, 0)

        def body2(k2, carry):
            k0 = k2 * 2

            @pl.when(k0 + 1 < per_w)
            def _():
                rd_issue(k0 + 1, 1)

            rd_wait(0)

            @pl.when(k0 >= 2)
            def _():
                wr_drain(0)

            transpose(0)
            wr_issue(k0, 0)

            @pl.when(k0 + 1 < per_w)
            def _():
                @pl.when(k0 + 2 < per_w)
                def _():
                    rd_issue(k0 + 2, 0)

                rd_wait(1)

                @pl.when(k0 >= 1)
                def _():
                    wr_drain(1)

                transpose(1)
                wr_issue(k0 + 1, 1)

            return carry

        lax.fori_loop(0, (per_w + 1) // 2, body2, 0)
        wr_drain(0)
        wr_drain(1)

        @pl.when(wid == 0)
        def _():
            pltpu.sync_copy(tabT_hbm.at[:, pl.ds(r512, RC)], b0)
            transpose(0)
            pltpu.sync_copy(t0, tl_hbm.at[pl.ds(r512 * D, RC * D)])

        @pl.when(wid == 17)
        def _():
            pltpu.sync_copy(tail_hbm, tt_v)
            pltpu.sync_copy(tt_v, tl_hbm.at[pl.ds(r64 * D, 64 * D)])

    return tr(tabT, tail_flat)


def _lookup(tl, idx2, fw, wrep, brep, B, L, V, D, NC, NS, mesh):
    NW = NC * NS
    BT = 512
    GB = B // BT
    n_blk = L * GB
    per_w = n_blk // NW
    NB = B * L * D

    @functools.partial(
        pl.kernel,
        mesh=mesh,
        out_type=jax.ShapeDtypeStruct((NB,), jnp.float32),
        compiler_params=pltpu.CompilerParams(
            use_tc_tiling_on_sc=False, needs_layout_passes=False),
        scratch_types=[
            pltpu.VMEM((BT,), jnp.int32),
            pltpu.VMEM((BT,), jnp.int32),
            pltpu.VMEM((BT,), jnp.float32),
            pltpu.VMEM((BT,), jnp.float32),
            pltpu.VMEM((BT, D), jnp.float32),
            pltpu.VMEM((BT, D), jnp.float32),
            pltpu.VMEM((BT * (D + 1),), jnp.float32),
            pltpu.VMEM((D // 8, BT * 8), jnp.float32),
            pltpu.VMEM((D // 8, BT * 8), jnp.float32),
            pltpu.VMEM((D * 16,), jnp.float32),
            pltpu.VMEM((D * 16,), jnp.float32),
            pltpu.SemaphoreType.DMA,
            pltpu.SemaphoreType.DMA,
            pltpu.SemaphoreType.DMA,
            pltpu.SemaphoreType.DMA,
            pltpu.SemaphoreType.DMA,
            pltpu.SemaphoreType.DMA,
            pltpu.SemaphoreType.DMA,
            pltpu.SemaphoreType.DMA,
        ],
    )
    def lk(tl_hbm, idx_hbm, fw_hbm, wrep_hbm, brep_hbm, out_hbm,
           i0, i1, f0, f1, r0, r1, rx, c0, c1, w_v, b_v,
           sem_x0, sem_x1, sem_r0, sem_r1, sem_f0, sem_f1, sem_c0, sem_c1):
        wid = lax.axis_index("s") * NC + lax.axis_index("c")
        pltpu.sync_copy(wrep_hbm, w_v)
        pltpu.sync_copy(brep_hbm, b_v)
        iota = lax.iota(jnp.int32, 16)
        idxs = [i0, i1]
        fvs = [f0, f1]
        rows = [r0, r1]
        chunks = [c0, c1]
        xsems = [sem_x0, sem_x1]
        rsems = [sem_r0, sem_r1]
        fsems = [sem_f0, sem_f1]
        csems = [sem_c0, sem_c1]

        def issue_idx(t, bi):
            j = t * NW + wid
            l = j // GB
            g = j % GB
            pltpu.async_copy(idx_hbm.at[pl.ds(l * B + g * BT, BT)],
                             idxs[bi], xsems[bi])

        QT = BT // 4
        HT = BT // 2

        def issue_gth(t, bi):
            pltpu.make_async_copy(idx_hbm.at[pl.ds(0, BT)], idxs[bi],
                                  xsems[bi]).wait()
            for p in range(4):
                pltpu.async_copy(
                    tl_hbm.at[idxs[bi].at[pl.ds(p * QT, QT)]],
                    rows[bi].at[pl.ds(p * QT, QT), :], rsems[bi])
            for p in range(2):
                pltpu.async_copy(
                    fw_hbm.at[idxs[bi].at[pl.ds(p * HT, HT)]],
                    fvs[bi].at[pl.ds(p * HT, HT)], fsems[bi])

        def wait_in(bi):
            for p in range(4):
                pltpu.make_async_copy(
                    tl_hbm.at[idxs[bi].at[pl.ds(0, QT)]],
                    rows[bi].at[pl.ds(0, QT), :], rsems[bi]).wait()
            for p in range(2):
                pltpu.make_async_copy(
                    fw_hbm.at[idxs[bi].at[pl.ds(0, HT)]],
                    fvs[bi].at[pl.ds(0, HT)], fsems[bi]).wait()

        def drain_out(bi):
            for dt in range(D // 8):
                pltpu.make_async_copy(chunks[bi].at[dt],
                                      out_hbm.at[pl.ds(0, BT * 8)],
                                      csems[bi]).wait()

        stride = D + 1
        i33 = iota * stride

        def compute(t, bi):
            j = t * NW + wid
            l = j // GB
            g = j % GB

            def rp_body(r, carry):
                rx[pl.ds(r * stride, 16)] = rows[bi][r, pl.ds(0, 16)]
                rx[pl.ds(r * stride + 16, 16)] = rows[bi][r, pl.ds(16, 16)]
                return carry

            lax.fori_loop(0, BT, rp_body, 0, unroll=8)
            for dt in range(D // 8):
                wv = [w_v[pl.ds((dt * 8 + ds) * 16, 16)] for ds in range(8)]
                bv = [b_v[pl.ds((dt * 8 + ds) * 16, 16)] for ds in range(8)]

                def u_body(u, carry2):
                    r_idx = u * 16 + iota
                    fvv = plsc.load_gather(fvs[bi], [r_idx])
                    rbase = u * (16 * stride)
                    off = (u // 8) * (128 * 8) + (u % 8) * 16
                    for ds in range(8):
                        d = dt * 8 + ds
                        src = plsc.load_gather(rx, [i33 + (rbase + d)])
                        chunks[bi][dt, pl.ds(off + ds * 128, 16)] = (
                            src + fvv * wv[ds] + bv[ds])
                    return carry2

                lax.fori_loop(0, BT // 16, u_body, 0, unroll=2)
            for dt in range(D // 8):
                pltpu.async_copy(
                    chunks[bi].at[dt],
                    out_hbm.at[pl.ds((l * (D // 8) + dt) * (B * 8) + g * BT * 8,
                                     BT * 8)],
                    csems[bi])

        issue_idx(0, 0)
        issue_idx(1, 1)
        issue_gth(0, 0)

        def body2(t2, carry):
            t = t2 * 2
            issue_gth(t + 1, 1)
            wait_in(0)

            @pl.when(t + 2 < per_w)
            def _():
                issue_idx(t + 2, 0)

            @pl.when(t >= 2)
            def _():
                drain_out(0)

            compute(t, 0)

            @pl.when(t + 2 < per_w)
            def _():
                issue_gth(t + 2, 0)

            wait_in(1)

            @pl.when(t + 3 < per_w)
            def _():
                issue_idx(t + 3, 1)

            @pl.when(t >= 1)
            def _():
                drain_out(1)

            compute(t + 1, 1)
            return carry

        lax.fori_loop(0, per_w // 2, body2, 0)
        drain_out(0)
        drain_out(1)

    return lk(tl, idx2, fw, wrep, brep)


def kernel(x, emb_table, freq_weights, freq_proj_w, freq_proj_b):
    B, L = x.shape
    V, D = emb_table.shape
    N = B * L

    info = plsc.get_sparse_core_info()
    NC, NS = info.num_cores, info.num_subcores
    mesh = plsc.VectorSubcoreMesh(core_axis_name="c", subcore_axis_name="s")

    tabT = jnp.transpose(emb_table)
    idx2 = jnp.transpose(x).reshape(N).astype(jnp.int32)
    wrep = jnp.repeat(0.1 * freq_proj_w[:, 0], 16).astype(jnp.float32)
    brep = jnp.repeat(0.1 * freq_proj_b, 16).astype(jnp.float32)
    tail_flat = emb_table[V - 64:].reshape(64 * D).astype(jnp.float32)

    tl_flat = _transpose_table(tabT, tail_flat, V, D, NC, NS, mesh)
    tl = tl_flat.reshape(V, D)
    o = _lookup(tl, idx2, freq_weights, wrep, brep, B, L, V, D, NC, NS, mesh)
    o5 = o.reshape(L, D // 8, B // 128, 8, 128)
    return jnp.transpose(o5, (2, 4, 0, 1, 3)).reshape(B, L, D)


# speedup vs baseline: 4.2618x; 1.0229x over previous
import functools

import jax
import jax.numpy as jnp
from jax import lax
from jax.experimental import pallas as pl
from jax.experimental.pallas import tpu as pltpu
from jax.experimental.pallas import tpu_sc as plsc


def _transpose_table(tabT, tail_flat, V, D, NC, NS, mesh):
    NW = NC * NS
    RC = 512
    n_main = 1952
    per_w = n_main // NW
    r512 = n_main * RC
    r64 = r512 + 512

    @functools.partial(
        pl.kernel,
        mesh=mesh,
        out_type=jax.ShapeDtypeStruct((V * D,), jnp.float32),
        compiler_params=pltpu.CompilerParams(
            use_tc_tiling_on_sc=True, needs_layout_passes=False),
        scratch_types=[
            pltpu.VMEM((D, RC), jnp.float32),
            pltpu.VMEM((D, RC), jnp.float32),
            pltpu.VMEM((D * (RC + 1),), jnp.float32),
            pltpu.VMEM((RC * D,), jnp.float32),
            pltpu.VMEM((RC * D,), jnp.float32),
            pltpu.VMEM((64 * D,), jnp.float32),
            pltpu.SemaphoreType.DMA,
            pltpu.SemaphoreType.DMA,
            pltpu.SemaphoreType.DMA,
            pltpu.SemaphoreType.DMA,
        ],
    )
    def tr(tabT_hbm, tail_hbm, tl_hbm, b0, b1, bx, t0, t1, tt_v,
           sem_i0, sem_i1, sem_w0, sem_w1):
        wid = lax.axis_index("s") * NC + lax.axis_index("c")
        iota = lax.iota(jnp.int32, 16)
        stride = RC + 1
        ilo = iota * stride
        ihi = ilo + 16 * stride
        bufs = [b0, b1]
        tls = [t0, t1]
        isems = [sem_i0, sem_i1]
        wsems = [sem_w0, sem_w1]

        def chunk_r0(k):
            return (k * NW + wid) * RC

        def rd_issue(k, bi):
            for ct in range(D // 8):
                pltpu.async_copy(
                    tabT_hbm.at[pl.ds(ct * 8, 8), pl.ds(chunk_r0(k), RC)],
                    bufs[bi].at[pl.ds(ct * 8, 8), :], isems[bi])

        def rd_wait(bi):
            for ct in range(D // 8):
                pltpu.make_async_copy(
                    tabT_hbm.at[pl.ds(0, 8), pl.ds(0, RC)],
                    bufs[bi].at[pl.ds(ct * 8, 8), :], isems[bi]).wait()

        def wr_issue(k, bi):
            pltpu.async_copy(tls[bi],
                             tl_hbm.at[pl.ds(chunk_r0(k) * D, RC * D)],
                             wsems[bi])

        def wr_drain(bi):
            pltpu.make_async_copy(tls[bi],
                                  tl_hbm.at[pl.ds(0, RC * D)],
                                  wsems[bi]).wait()

        def transpose(bi):
            def rp_body(c, carry):
                for rg in range(RC // 16):
                    bx[pl.ds(c * stride + rg * 16, 16)] = (
                        bufs[bi][c, pl.ds(rg * 16, 16)])
                return carry

            lax.fori_loop(0, D, rp_body, 0, unroll=4)

            def row_body(r, carry):
                tls[bi][pl.ds(r * D, 16)] = plsc.load_gather(bx, [ilo + r])
                tls[bi][pl.ds(r * D + 16, 16)] = plsc.load_gather(bx, [ihi + r])
                return carry

            lax.fori_loop(0, RC, row_body, 0, unroll=16)

        rd_issue(0, 0)

        def body2(k2, carry):
            k0 = k2 * 2

            @pl.when(k0 + 1 < per_w)
            def _():
                rd_issue(k0 + 1, 1)

            rd_wait(0)

            @pl.when(k0 >= 2)
            def _():
                wr_drain(0)

            transpose(0)
            wr_issue(k0, 0)

            @pl.when(k0 + 1 < per_w)
            def _():
                @pl.when(k0 + 2 < per_w)
                def _():
                    rd_issue(k0 + 2, 0)

                rd_wait(1)

                @pl.when(k0 >= 1)
                def _():
                    wr_drain(1)

                transpose(1)
                wr_issue(k0 + 1, 1)

            return carry

        lax.fori_loop(0, (per_w + 1) // 2, body2, 0)
        wr_drain(0)
        wr_drain(1)

        @pl.when(wid == 0)
        def _():
            pltpu.sync_copy(tabT_hbm.at[:, pl.ds(r512, RC)], b0)
            transpose(0)
            pltpu.sync_copy(t0, tl_hbm.at[pl.ds(r512 * D, RC * D)])

        @pl.when(wid == 17)
        def _():
            pltpu.sync_copy(tail_hbm, tt_v)
            pltpu.sync_copy(tt_v, tl_hbm.at[pl.ds(r64 * D, 64 * D)])

    return tr(tabT, tail_flat)


def _lookup(tl, idx2, fw, wrep, brep, B, L, V, D, NC, NS, mesh):
    NW = NC * NS
    BT = 512
    GB = B // BT
    n_blk = L * GB
    per_w = n_blk // NW
    NB = B * L * D

    @functools.partial(
        pl.kernel,
        mesh=mesh,
        out_type=jax.ShapeDtypeStruct((NB,), jnp.float32),
        compiler_params=pltpu.CompilerParams(
            use_tc_tiling_on_sc=False, needs_layout_passes=False),
        scratch_types=[
            pltpu.VMEM((BT,), jnp.int32),
            pltpu.VMEM((BT,), jnp.int32),
            pltpu.VMEM((BT,), jnp.float32),
            pltpu.VMEM((BT,), jnp.float32),
            pltpu.VMEM((BT, D), jnp.float32),
            pltpu.VMEM((BT, D), jnp.float32),
            pltpu.VMEM((BT * (D + 1),), jnp.float32),
            pltpu.VMEM((D // 8, BT * 8), jnp.float32),
            pltpu.VMEM((D // 8, BT * 8), jnp.float32),
            pltpu.VMEM((D * 16,), jnp.float32),
            pltpu.VMEM((D * 16,), jnp.float32),
            pltpu.SemaphoreType.DMA,
            pltpu.SemaphoreType.DMA,
            pltpu.SemaphoreType.DMA,
            pltpu.SemaphoreType.DMA,
            pltpu.SemaphoreType.DMA,
            pltpu.SemaphoreType.DMA,
            pltpu.SemaphoreType.DMA,
            pltpu.SemaphoreType.DMA,
        ],
    )
    def lk(tl_hbm, idx_hbm, fw_hbm, wrep_hbm, brep_hbm, out_hbm,
           i0, i1, f0, f1, r0, r1, rx, c0, c1, w_v, b_v,
           sem_x0, sem_x1, sem_r0, sem_r1, sem_f0, sem_f1, sem_c0, sem_c1):
        wid = lax.axis_index("s") * NC + lax.axis_index("c")
        pltpu.sync_copy(wrep_hbm, w_v)
        pltpu.sync_copy(brep_hbm, b_v)
        iota = lax.iota(jnp.int32, 16)
        idxs = [i0, i1]
        fvs = [f0, f1]
        rows = [r0, r1]
        chunks = [c0, c1]
        xsems = [sem_x0, sem_x1]
        rsems = [sem_r0, sem_r1]
        fsems = [sem_f0, sem_f1]
        csems = [sem_c0, sem_c1]

        def issue_idx(t, bi):
            j = t * NW + wid
            l = j // GB
            g = j % GB
            pltpu.async_copy(idx_hbm.at[pl.ds(l * B + g * BT, BT)],
                             idxs[bi], xsems[bi])

        QT = BT // 4
        HT = BT // 2

        def issue_gth(t, bi):
            pltpu.make_async_copy(idx_hbm.at[pl.ds(0, BT)], idxs[bi],
                                  xsems[bi]).wait()
            for p in range(4):
                pltpu.async_copy(
                    tl_hbm.at[idxs[bi].at[pl.ds(p * QT, QT)]],
                    rows[bi].at[pl.ds(p * QT, QT), :], rsems[bi])
            for p in range(2):
                pltpu.async_copy(
                    fw_hbm.at[idxs[bi].at[pl.ds(p * HT, HT)]],
                    fvs[bi].at[pl.ds(p * HT, HT)], fsems[bi])

        def wait_in(bi):
            for p in range(4):
                pltpu.make_async_copy(
                    tl_hbm.at[idxs[bi].at[pl.ds(0, QT)]],
                    rows[bi].at[pl.ds(0, QT), :], rsems[bi]).wait()
            for p in range(2):
                pltpu.make_async_copy(
                    fw_hbm.at[idxs[bi].at[pl.ds(0, HT)]],
                    fvs[bi].at[pl.ds(0, HT)], fsems[bi]).wait()

        def drain_out(bi):
            for dt in range(D // 8):
                pltpu.make_async_copy(chunks[bi].at[dt],
                                      out_hbm.at[pl.ds(0, BT * 8)],
                                      csems[bi]).wait()

        stride = D + 1
        i33 = iota * stride

        def compute(t, bi):
            j = t * NW + wid
            l = j // GB
            g = j % GB

            def rp_body(r, carry):
                rx[pl.ds(r * stride, 16)] = rows[bi][r, pl.ds(0, 16)]
                rx[pl.ds(r * stride + 16, 16)] = rows[bi][r, pl.ds(16, 16)]
                return carry

            lax.fori_loop(0, BT, rp_body, 0, unroll=16)
            for dt in range(D // 8):
                wv = [w_v[pl.ds((dt * 8 + ds) * 16, 16)] for ds in range(8)]
                bv = [b_v[pl.ds((dt * 8 + ds) * 16, 16)] for ds in range(8)]

                def u_body(u, carry2):
                    r_idx = u * 16 + iota
                    fvv = plsc.load_gather(fvs[bi], [r_idx])
                    rbase = u * (16 * stride)
                    off = (u // 8) * (128 * 8) + (u % 8) * 16
                    for ds in range(8):
                        d = dt * 8 + ds
                        src = plsc.load_gather(rx, [i33 + (rbase + d)])
                        chunks[bi][dt, pl.ds(off + ds * 128, 16)] = (
                            src + fvv * wv[ds] + bv[ds])
                    return carry2

                lax.fori_loop(0, BT // 16, u_body, 0, unroll=4)
            for dt in range(D // 8):
                pltpu.async_copy(
                    chunks[bi].at[dt],
                    out_hbm.at[pl.ds((l * (D // 8) + dt) * (B * 8) + g * BT * 8,
                                     BT * 8)],
                    csems[bi])

        issue_idx(0, 0)
        issue_idx(1, 1)
        issue_gth(0, 0)

        def body2(t2, carry):
            t = t2 * 2
            issue_gth(t + 1, 1)
            wait_in(0)

            @pl.when(t + 2 < per_w)
            def _():
                issue_idx(t + 2, 0)

            @pl.when(t >= 2)
            def _():
                drain_out(0)

            compute(t, 0)

            @pl.when(t + 2 < per_w)
            def _():
                issue_gth(t + 2, 0)

            wait_in(1)

            @pl.when(t + 3 < per_w)
            def _():
                issue_idx(t + 3, 1)

            @pl.when(t >= 1)
            def _():
                drain_out(1)

            compute(t + 1, 1)
            return carry

        lax.fori_loop(0, per_w // 2, body2, 0)
        drain_out(0)
        drain_out(1)

    return lk(tl, idx2, fw, wrep, brep)


def kernel(x, emb_table, freq_weights, freq_proj_w, freq_proj_b):
    B, L = x.shape
    V, D = emb_table.shape
    N = B * L

    info = plsc.get_sparse_core_info()
    NC, NS = info.num_cores, info.num_subcores
    mesh = plsc.VectorSubcoreMesh(core_axis_name="c", subcore_axis_name="s")

    tabT = jnp.transpose(emb_table)
    idx2 = jnp.transpose(x).reshape(N).astype(jnp.int32)
    wrep = jnp.repeat(0.1 * freq_proj_w[:, 0], 16).astype(jnp.float32)
    brep = jnp.repeat(0.1 * freq_proj_b, 16).astype(jnp.float32)
    tail_flat = emb_table[V - 64:].reshape(64 * D).astype(jnp.float32)

    tl_flat = _transpose_table(tabT, tail_flat, V, D, NC, NS, mesh)
    tl = tl_flat.reshape(V, D)
    o = _lookup(tl, idx2, freq_weights, wrep, brep, B, L, V, D, NC, NS, mesh)
    o5 = o.reshape(L, D // 8, B // 128, 8, 128)
    return jnp.transpose(o5, (2, 4, 0, 1, 3)).reshape(B, L, D)


# speedup vs baseline: 4.3828x; 1.0284x over previous
import functools

import jax
import jax.numpy as jnp
from jax import lax
from jax.experimental import pallas as pl
from jax.experimental.pallas import tpu as pltpu
from jax.experimental.pallas import tpu_sc as plsc


def _transpose_table(tabT, tail_flat, V, D, NC, NS, mesh):
    NW = NC * NS
    RC = 512
    n_main = 1952
    per_w = n_main // NW
    r512 = n_main * RC
    r64 = r512 + 512

    @functools.partial(
        pl.kernel,
        mesh=mesh,
        out_type=jax.ShapeDtypeStruct((V * D,), jnp.float32),
        compiler_params=pltpu.CompilerParams(
            use_tc_tiling_on_sc=True, needs_layout_passes=False),
        scratch_types=[
            pltpu.VMEM((D, RC), jnp.float32),
            pltpu.VMEM((D, RC), jnp.float32),
            pltpu.VMEM((D * (RC + 1),), jnp.float32),
            pltpu.VMEM((RC * D,), jnp.float32),
            pltpu.VMEM((RC * D,), jnp.float32),
            pltpu.VMEM((64 * D,), jnp.float32),
            pltpu.SemaphoreType.DMA,
            pltpu.SemaphoreType.DMA,
            pltpu.SemaphoreType.DMA,
            pltpu.SemaphoreType.DMA,
        ],
    )
    def tr(tabT_hbm, tail_hbm, tl_hbm, b0, b1, bx, t0, t1, tt_v,
           sem_i0, sem_i1, sem_w0, sem_w1):
        wid = lax.axis_index("s") * NC + lax.axis_index("c")
        iota = lax.iota(jnp.int32, 16)
        stride = RC + 1
        ilo = iota * stride
        ihi = ilo + 16 * stride
        bufs = [b0, b1]
        tls = [t0, t1]
        isems = [sem_i0, sem_i1]
        wsems = [sem_w0, sem_w1]

        def chunk_r0(k):
            return (k * NW + wid) * RC

        def rd_issue(k, bi):
            for ct in range(D // 8):
                pltpu.async_copy(
                    tabT_hbm.at[pl.ds(ct * 8, 8), pl.ds(chunk_r0(k), RC)],
                    bufs[bi].at[pl.ds(ct * 8, 8), :], isems[bi])

        def rd_wait(bi):
            for ct in range(D // 8):
                pltpu.make_async_copy(
                    tabT_hbm.at[pl.ds(0, 8), pl.ds(0, RC)],
                    bufs[bi].at[pl.ds(ct * 8, 8), :], isems[bi]).wait()

        def wr_issue(k, bi):
            pltpu.async_copy(tls[bi],
                             tl_hbm.at[pl.ds(chunk_r0(k) * D, RC * D)],
                             wsems[bi])

        def wr_drain(bi):
            pltpu.make_async_copy(tls[bi],
                                  tl_hbm.at[pl.ds(0, RC * D)],
                                  wsems[bi]).wait()

        def transpose(bi):
            def rp_body(c, carry):
                for rg in range(RC // 16):
                    bx[pl.ds(c * stride + rg * 16, 16)] = (
                        bufs[bi][c, pl.ds(rg * 16, 16)])
                return carry

            lax.fori_loop(0, D, rp_body, 0, unroll=8)

            def row_body(r, carry):
                tls[bi][pl.ds(r * D, 16)] = plsc.load_gather(bx, [ilo + r])
                tls[bi][pl.ds(r * D + 16, 16)] = plsc.load_gather(bx, [ihi + r])
                return carry

            lax.fori_loop(0, RC, row_body, 0, unroll=32)

        rd_issue(0, 0)

        def body2(k2, carry):
            k0 = k2 * 2

            @pl.when(k0 + 1 < per_w)
            def _():
                rd_issue(k0 + 1, 1)

            rd_wait(0)

            @pl.when(k0 >= 2)
            def _():
                wr_drain(0)

            transpose(0)
            wr_issue(k0, 0)

            @pl.when(k0 + 1 < per_w)
            def _():
                @pl.when(k0 + 2 < per_w)
                def _():
                    rd_issue(k0 + 2, 0)

                rd_wait(1)

                @pl.when(k0 >= 1)
                def _():
                    wr_drain(1)

                transpose(1)
                wr_issue(k0 + 1, 1)

            return carry

        lax.fori_loop(0, (per_w + 1) // 2, body2, 0)
        wr_drain(0)
        wr_drain(1)

        @pl.when(wid == 0)
        def _():
            pltpu.sync_copy(tabT_hbm.at[:, pl.ds(r512, RC)], b0)
            transpose(0)
            pltpu.sync_copy(t0, tl_hbm.at[pl.ds(r512 * D, RC * D)])

        @pl.when(wid == 17)
        def _():
            pltpu.sync_copy(tail_hbm, tt_v)
            pltpu.sync_copy(tt_v, tl_hbm.at[pl.ds(r64 * D, 64 * D)])

    return tr(tabT, tail_flat)


def _lookup(tl, idx2, fw, wrep, brep, B, L, V, D, NC, NS, mesh):
    NW = NC * NS
    BT = 512
    GB = B // BT
    n_blk = L * GB
    per_w = n_blk // NW
    NB = B * L * D

    @functools.partial(
        pl.kernel,
        mesh=mesh,
        out_type=jax.ShapeDtypeStruct((NB,), jnp.float32),
        compiler_params=pltpu.CompilerParams(
            use_tc_tiling_on_sc=False, needs_layout_passes=False),
        scratch_types=[
            pltpu.VMEM((BT,), jnp.int32),
            pltpu.VMEM((BT,), jnp.int32),
            pltpu.VMEM((BT,), jnp.float32),
            pltpu.VMEM((BT,), jnp.float32),
            pltpu.VMEM((BT, D), jnp.float32),
            pltpu.VMEM((BT, D), jnp.float32),
            pltpu.VMEM((BT * (D + 1),), jnp.float32),
            pltpu.VMEM((D // 8, BT * 8), jnp.float32),
            pltpu.VMEM((D // 8, BT * 8), jnp.float32),
            pltpu.VMEM((D * 16,), jnp.float32),
            pltpu.VMEM((D * 16,), jnp.float32),
            pltpu.SemaphoreType.DMA,
            pltpu.SemaphoreType.DMA,
            pltpu.SemaphoreType.DMA,
            pltpu.SemaphoreType.DMA,
            pltpu.SemaphoreType.DMA,
            pltpu.SemaphoreType.DMA,
            pltpu.SemaphoreType.DMA,
            pltpu.SemaphoreType.DMA,
        ],
    )
    def lk(tl_hbm, idx_hbm, fw_hbm, wrep_hbm, brep_hbm, out_hbm,
           i0, i1, f0, f1, r0, r1, rx, c0, c1, w_v, b_v,
           sem_x0, sem_x1, sem_r0, sem_r1, sem_f0, sem_f1, sem_c0, sem_c1):
        wid = lax.axis_index("s") * NC + lax.axis_index("c")
        pltpu.sync_copy(wrep_hbm, w_v)
        pltpu.sync_copy(brep_hbm, b_v)
        iota = lax.iota(jnp.int32, 16)
        idxs = [i0, i1]
        fvs = [f0, f1]
        rows = [r0, r1]
        chunks = [c0, c1]
        xsems = [sem_x0, sem_x1]
        rsems = [sem_r0, sem_r1]
        fsems = [sem_f0, sem_f1]
        csems = [sem_c0, sem_c1]

        def issue_idx(t, bi):
            j = t * NW + wid
            l = j // GB
            g = j % GB
            pltpu.async_copy(idx_hbm.at[pl.ds(l * B + g * BT, BT)],
                             idxs[bi], xsems[bi])

        QT = BT // 4
        HT = BT // 2

        def issue_gth(t, bi):
            pltpu.make_async_copy(idx_hbm.at[pl.ds(0, BT)], idxs[bi],
                                  xsems[bi]).wait()
            for p in range(4):
                pltpu.async_copy(
                    tl_hbm.at[idxs[bi].at[pl.ds(p * QT, QT)]],
                    rows[bi].at[pl.ds(p * QT, QT), :], rsems[bi])
            for p in range(2):
                pltpu.async_copy(
                    fw_hbm.at[idxs[bi].at[pl.ds(p * HT, HT)]],
                    fvs[bi].at[pl.ds(p * HT, HT)], fsems[bi])

        def wait_in(bi):
            for p in range(4):
                pltpu.make_async_copy(
                    tl_hbm.at[idxs[bi].at[pl.ds(0, QT)]],
                    rows[bi].at[pl.ds(0, QT), :], rsems[bi]).wait()
            for p in range(2):
                pltpu.make_async_copy(
                    fw_hbm.at[idxs[bi].at[pl.ds(0, HT)]],
                    fvs[bi].at[pl.ds(0, HT)], fsems[bi]).wait()

        def drain_out(bi):
            for dt in range(D // 8):
                pltpu.make_async_copy(chunks[bi].at[dt],
                                      out_hbm.at[pl.ds(0, BT * 8)],
                                      csems[bi]).wait()

        stride = D + 1
        i33 = iota * stride

        def compute(t, bi):
            j = t * NW + wid
            l = j // GB
            g = j % GB

            def rp_body(r, carry):
                rx[pl.ds(r * stride, 16)] = rows[bi][r, pl.ds(0, 16)]
                rx[pl.ds(r * stride + 16, 16)] = rows[bi][r, pl.ds(16, 16)]
                return carry

            lax.fori_loop(0, BT, rp_body, 0, unroll=16)
            for dt in range(D // 8):
                wv = [w_v[pl.ds((dt * 8 + ds) * 16, 16)] for ds in range(8)]
                bv = [b_v[pl.ds((dt * 8 + ds) * 16, 16)] for ds in range(8)]

                def u_body(u, carry2):
                    r_idx = u * 16 + iota
                    fvv = plsc.load_gather(fvs[bi], [r_idx])
                    rbase = u * (16 * stride)
                    off = (u // 8) * (128 * 8) + (u % 8) * 16
                    for ds in range(8):
                        d = dt * 8 + ds
                        src = plsc.load_gather(rx, [i33 + (rbase + d)])
                        chunks[bi][dt, pl.ds(off + ds * 128, 16)] = (
                            src + fvv * wv[ds] + bv[ds])
                    return carry2

                lax.fori_loop(0, BT // 16, u_body, 0, unroll=4)
            for dt in range(D // 8):
                pltpu.async_copy(
                    chunks[bi].at[dt],
                    out_hbm.at[pl.ds((l * (D // 8) + dt) * (B * 8) + g * BT * 8,
                                     BT * 8)],
                    csems[bi])

        issue_idx(0, 0)
        issue_idx(1, 1)
        issue_gth(0, 0)

        def body2(t2, carry):
            t = t2 * 2
            issue_gth(t + 1, 1)
            wait_in(0)

            @pl.when(t + 2 < per_w)
            def _():
                issue_idx(t + 2, 0)

            @pl.when(t >= 2)
            def _():
                drain_out(0)

            compute(t, 0)

            @pl.when(t + 2 < per_w)
            def _():
                issue_gth(t + 2, 0)

            wait_in(1)

            @pl.when(t + 3 < per_w)
            def _():
                issue_idx(t + 3, 1)

            @pl.when(t >= 1)
            def _():
                drain_out(1)

            compute(t + 1, 1)
            return carry

        lax.fori_loop(0, per_w // 2, body2, 0)
        drain_out(0)
        drain_out(1)

    return lk(tl, idx2, fw, wrep, brep)


def kernel(x, emb_table, freq_weights, freq_proj_w, freq_proj_b):
    B, L = x.shape
    V, D = emb_table.shape
    N = B * L

    info = plsc.get_sparse_core_info()
    NC, NS = info.num_cores, info.num_subcores
    mesh = plsc.VectorSubcoreMesh(core_axis_name="c", subcore_axis_name="s")

    tabT = jnp.transpose(emb_table)
    idx2 = jnp.transpose(x).reshape(N).astype(jnp.int32)
    wrep = jnp.repeat(0.1 * freq_proj_w[:, 0], 16).astype(jnp.float32)
    brep = jnp.repeat(0.1 * freq_proj_b, 16).astype(jnp.float32)
    tail_flat = emb_table[V - 64:].reshape(64 * D).astype(jnp.float32)

    tl_flat = _transpose_table(tabT, tail_flat, V, D, NC, NS, mesh)
    tl = tl_flat.reshape(V, D)
    o = _lookup(tl, idx2, freq_weights, wrep, brep, B, L, V, D, NC, NS, mesh)
    o5 = o.reshape(L, D // 8, B // 128, 8, 128)
    return jnp.transpose(o5, (2, 4, 0, 1, 3)).reshape(B, L, D)
